# baseline (speedup 1.0000x reference)
import functools

import jax
import jax.numpy as jnp
from jax import lax
from jax.experimental import pallas as pl
from jax.experimental.pallas import tpu as pltpu
from jax.experimental.pallas import tpu_sc as plsc

N, D = 2048, 768
H, DH = 12, 64
M, K = 32768, 32
FF = 4 * D
SCALE = DH ** -0.5
NT = 256
QT = 512


def _lnf(x, g, b):
    m = x.mean(-1, keepdims=True)
    v = ((x - m) ** 2).mean(-1, keepdims=True)
    return (x - m) / jnp.sqrt(v + 1e-5) * g + b


def _gelu(x):
    c = 0.7978845608028654
    return 0.5 * x * (1.0 + jnp.tanh(c * (x + 0.044715 * x * x * x)))


def _dot(a, b):
    return jax.lax.dot_general(a, b, (((1,), (0,)), ((), ())),
                               preferred_element_type=jnp.float32)


def _dot_t(a, b):
    return jax.lax.dot_general(a, b, (((1,), (1,)), ((), ())),
                               preferred_element_type=jnp.float32)


def _qkv_body(x_ref, g_ref, b_ref, wq_ref, wk_ref, wv_ref,
              q_ref, k_ref, v_ref):
    y = _lnf(x_ref[...], g_ref[...], b_ref[...])
    q_ref[...] = _dot(y, wq_ref[...])
    k_ref[...] = _dot(y, wk_ref[...])
    v_ref[...] = _dot(y, wv_ref[...])


def _qkv(x, g, b, wq, wk, wv):
    grid = (N // NT,)
    bs_x = pl.BlockSpec((NT, D), lambda i: (i, 0))
    bs_vec = pl.BlockSpec((1, D), lambda i: (0, 0))
    bs_w = pl.BlockSpec((D, D), lambda i: (0, 0))
    bs_o = pl.BlockSpec((NT, D), lambda i: (i, 0))
    out = pl.pallas_call(
        _qkv_body,
        grid=grid,
        in_specs=[bs_x, bs_vec, bs_vec, bs_w, bs_w, bs_w],
        out_specs=[bs_o, bs_o, bs_o],
        out_shape=[jax.ShapeDtypeStruct((N, D), jnp.float32)] * 3,
    )(x, g.reshape(1, D), b.reshape(1, D), wq, wk, wv)
    return out


def _attn0_body(q_ref, k_ref, v_ref, o_ref):
    qi = pl.program_id(1)
    q = q_ref[0]
    s = _dot_t(q, k_ref[0]) * SCALE
    row = qi * QT + jax.lax.broadcasted_iota(jnp.int32, (QT, N), 0)
    col = jax.lax.broadcasted_iota(jnp.int32, (QT, N), 1)
    s = jnp.where(row >= col, s, -1e9)
    m = jnp.max(s, axis=-1, keepdims=True)
    e = jnp.exp(s - m)
    a = e / jnp.sum(e, axis=-1, keepdims=True)
    o_ref[0] = _dot(a, v_ref[0])


def _attn0(qh, kh, vh):
    grid = (H, N // QT)
    bs_q = pl.BlockSpec((1, QT, DH), lambda h, i: (h, i, 0))
    bs_kv = pl.BlockSpec((1, N, DH), lambda h, i: (h, 0, 0))
    return pl.pallas_call(
        _attn0_body,
        grid=grid,
        in_specs=[bs_q, bs_kv, bs_kv],
        out_specs=bs_q,
        out_shape=jax.ShapeDtypeStruct((H, N, DH), jnp.float32),
    )(qh, kh, vh)


def _proj_body(h_ref, a_ref, w_ref, o_ref):
    o_ref[...] = h_ref[...] + _dot(a_ref[...], w_ref[...])


def _proj_residual(h, a, wo):
    grid = (N // NT,)
    bs = pl.BlockSpec((NT, D), lambda i: (i, 0))
    bs_w = pl.BlockSpec((D, D), lambda i: (0, 0))
    return pl.pallas_call(
        _proj_body,
        grid=grid,
        in_specs=[bs, bs, bs_w],
        out_specs=bs,
        out_shape=jax.ShapeDtypeStruct((N, D), jnp.float32),
    )(h, a, wo)


def _ffn_body(h_ref, g_ref, b_ref, w1_ref, b1_ref, w2_ref, b2_ref, o_ref):
    h = h_ref[...]
    y = _lnf(h, g_ref[...], b_ref[...])
    u = _gelu(_dot(y, w1_ref[...]) + b1_ref[...])
    o_ref[...] = h + _dot(u, w2_ref[...]) + b2_ref[...]


def _ffn(h, g, b, w1, b1, w2, b2):
    grid = (N // NT,)
    bs = pl.BlockSpec((NT, D), lambda i: (i, 0))
    bs_vd = pl.BlockSpec((1, D), lambda i: (0, 0))
    bs_vf = pl.BlockSpec((1, FF), lambda i: (0, 0))
    bs_w1 = pl.BlockSpec((D, FF), lambda i: (0, 0))
    bs_w2 = pl.BlockSpec((FF, D), lambda i: (0, 0))
    return pl.pallas_call(
        _ffn_body,
        grid=grid,
        in_specs=[bs, bs_vd, bs_vd, bs_w1, bs_vf, bs_w2, bs_vd],
        out_specs=bs,
        out_shape=jax.ShapeDtypeStruct((N, D), jnp.float32),
    )(h, g.reshape(1, D), b.reshape(1, D), w1, b1.reshape(1, FF),
      w2, b2.reshape(1, D))


def _ln_body(h_ref, g_ref, b_ref, o_ref):
    o_ref[...] = _lnf(h_ref[...], g_ref[...], b_ref[...])


def _final_ln(h, g, b):
    grid = (N // NT,)
    bs = pl.BlockSpec((NT, D), lambda i: (i, 0))
    bs_vec = pl.BlockSpec((1, D), lambda i: (0, 0))
    return pl.pallas_call(
        _ln_body,
        grid=grid,
        in_specs=[bs, bs_vec, bs_vec],
        out_specs=bs,
        out_shape=jax.ShapeDtypeStruct((N, D), jnp.float32),
    )(h, g.reshape(1, D), b.reshape(1, D))


def _memnorm_body(mk_ref, mv_ref, mkn_ref, mvp_ref):
    mk = mk_ref[...]
    nrm = jnp.sqrt(jnp.sum(mk * mk, axis=-1, keepdims=True)) + 1e-8
    mkn_ref[...] = mk / nrm
    mv = mv_ref[...]
    mvp_ref[...] = jnp.concatenate(
        [mv, nrm, jnp.zeros_like(mv[:, :DH - 1])], axis=-1)


def _memnorm(mem_k, mem_v):
    MT = 4096
    grid = (M // MT,)
    bs = pl.BlockSpec((MT, DH), lambda i: (i, 0))
    bs_p = pl.BlockSpec((MT, 2 * DH), lambda i: (i, 0))
    return pl.pallas_call(
        _memnorm_body,
        grid=grid,
        in_specs=[bs, bs],
        out_specs=[bs, bs_p],
        out_shape=[jax.ShapeDtypeStruct((M, DH), jnp.float32),
                   jax.ShapeDtypeStruct((M, 2 * DH), jnp.float32)],
    )(mem_k, mem_v)


def _qnorm_body(q_ref, qn_ref, qf_ref):
    q = q_ref[0]
    nrm = jnp.sqrt(jnp.sum(q * q, axis=-1, keepdims=True)) + 1e-8
    qn_ref[0] = q / nrm
    qf_ref[0] = nrm


def _qnorm(qh):
    grid = (H,)
    bs = pl.BlockSpec((1, N, DH), lambda h: (h, 0, 0))
    bs_n = pl.BlockSpec((1, N, 1), lambda h: (h, 0, 0))
    return pl.pallas_call(
        _qnorm_body,
        grid=grid,
        in_specs=[bs],
        out_specs=[bs, bs_n],
        out_shape=[jax.ShapeDtypeStruct((H, N, DH), jnp.float32),
                   jax.ShapeDtypeStruct((H, N, 1), jnp.float32)],
    )(qh)


MT_S = 2048
GBLK = 128
SBLK = 32
NBLK = M // SBLK
RR = H * N


def _scores_body(qn_ref, mkn_ref, s_ref, bm_ref):
    q = qn_ref[0]
    parts = []
    for j in range(MT_S // GBLK):
        s_j = _dot_t(q, mkn_ref[pl.ds(j * GBLK, GBLK), :])
        s_ref[:, pl.ds(j * GBLK, GBLK)] = s_j
        for u in range(GBLK // SBLK):
            parts.append(jnp.max(s_j[:, u * SBLK:(u + 1) * SBLK], axis=-1,
                                 keepdims=True))
    bm_ref[0] = jnp.concatenate(parts, axis=-1)


def _scores(qn, mkn, hg):
    rows = hg * N
    grid = (hg, M // MT_S)
    bs_q = pl.BlockSpec((1, N, DH), lambda h, m: (h, 0, 0))
    bs_m = pl.BlockSpec((MT_S, DH), lambda h, m: (m, 0))
    bs_s = pl.BlockSpec((N, MT_S), lambda h, m: (h, m))
    bs_b = pl.BlockSpec((1, N, MT_S // SBLK), lambda h, m: (m, h, 0))
    s2, bm3 = pl.pallas_call(
        _scores_body,
        grid=grid,
        in_specs=[bs_q, bs_m],
        out_specs=[bs_s, bs_b],
        out_shape=[jax.ShapeDtypeStruct((rows, M), jnp.float32),
                   jax.ShapeDtypeStruct((M // MT_S, rows, MT_S // SBLK),
                                        jnp.float32)],
    )(qn, mkn)
    bm = bm3.transpose(1, 0, 2).reshape(rows, NBLK)
    return s2.reshape(rows, M // GBLK, GBLK), bm


def _memattn_body(q_ref, k_ref, v_ref, hs_ref, qf_ref, gv_ref, gate_ref,
                  o_ref):
    qi = pl.program_id(1)
    q = q_ref[0]
    s = _dot_t(q, k_ref[0]) * SCALE
    row = qi * QT + jax.lax.broadcasted_iota(jnp.int32, (QT, N), 0)
    col = jax.lax.broadcasted_iota(jnp.int32, (QT, N), 1)
    s = jnp.where(row >= col, s, -1e9)
    sim_r = hs_ref[0] * qf_ref[0] * SCALE
    m = jnp.maximum(jnp.max(s, axis=-1, keepdims=True),
                    jnp.max(sim_r, axis=-1, keepdims=True))
    e_loc = jnp.exp(s - m)
    e_mem = jnp.exp(sim_r - m)
    denom = jnp.sum(e_loc, axis=-1, keepdims=True) + \
        jnp.sum(e_mem, axis=-1, keepdims=True)
    a_loc = e_loc / denom
    a_mem = e_mem / denom
    out_loc = _dot(a_loc, v_ref[0])
    gv = gv_ref[0]
    acc = a_mem[:, 0:1] * gv[:, 0, :DH]
    for kk in range(1, K):
        acc = acc + a_mem[:, kk:kk + 1] * gv[:, kk, :DH]
    g = gate_ref[0, 0, 0]
    o_ref[0] = g * acc + (1.0 - g) * out_loc


def _memattn(qh, kh, vh, halfsim, qfac, gv, gate):
    hg = qh.shape[0]
    grid = (hg, N // QT)
    bs_q = pl.BlockSpec((1, QT, DH), lambda h, i: (h, i, 0))
    bs_kv = pl.BlockSpec((1, N, DH), lambda h, i: (h, 0, 0))
    bs_hs = pl.BlockSpec((1, QT, K), lambda h, i: (h, i, 0))
    bs_qf = pl.BlockSpec((1, QT, 1), lambda h, i: (h, i, 0))
    bs_gv = pl.BlockSpec((1, QT, K, 2 * DH), lambda h, i: (h, i, 0, 0))
    bs_g = pl.BlockSpec((1, 1, 1), lambda h, i: (h, 0, 0))
    return pl.pallas_call(
        _memattn_body,
        grid=grid,
        in_specs=[bs_q, bs_kv, bs_kv, bs_hs, bs_qf, bs_gv, bs_g],
        out_specs=bs_q,
        out_shape=jax.ShapeDtypeStruct((hg, N, DH), jnp.float32),
    )(qh, kh, vh, halfsim, qfac, gv, gate)


R = H * N
NW = 32
RPW = R // NW
BATCH = 8
NBATCH = RPW // BATCH
NPAIR = M // GBLK
CMAX = 64
CSLOT = 80
SELCAP = 240
NEG = -3.0e38


def _sc_topk_body(rpw, nbatch, scores_hbm, bm_hbm, memv_hbm, hs_hbm, gv_hbm,
                  bmb, gidx, gath, candb, selv, selg, hsst, mvidx,
                  gvst, ccount_s, thr_s, sem_g, sem_v):
    nc = 2
    wid = lax.axis_index("s") * nc + lax.axis_index("c")
    base_row = wid * rpw
    iota = lax.iota(jnp.int32, 16)
    kncol = jnp.full((16,), DH, jnp.int32)

    def popcnt(m):
        return plsc.all_reduce_population_count(m)[0]

    def count_ge(i, t):
        c = jnp.int32(0)
        for j in range(NBLK // 16):
            c = c + popcnt(bmb[i, pl.ds(j * 16, 16)] >= t)
        return c

    def batch_body(b, carry):
        row0 = base_row + b * BATCH
        pltpu.sync_copy(bm_hbm.at[pl.ds(row0, BATCH)], bmb)
        zv = jnp.zeros((16,), jnp.int32)
        for j in range(BATCH * CSLOT // 16):
            candb[pl.ds(j * 16, 16)] = zv

        def p1_row(i, c1):
            mh = []
            for half in range(2):
                m = bmb[i, pl.ds(half * (NBLK // 2), 16)]
                for j in range(1, NBLK // 32):
                    m = jnp.maximum(
                        m, bmb[i, pl.ds(half * (NBLK // 2) + j * 16, 16)])
                mh.append(m)
            t0 = jnp.sort(jnp.minimum(mh[0], mh[1]))[0]
            rmax = jnp.sort(jnp.maximum(mh[0], mh[1]))[15]
            c0 = count_ge(i, t0)

            def w_cond(st):
                return jnp.logical_and(st[3] > 48, st[4] < 16)

            def w_body(st):
                lo, hi, t, c, it = st
                mid = 0.5 * (lo + hi)
                cm = count_ge(i, mid)
                ok = cm >= K
                return (jnp.where(ok, mid, lo), jnp.where(ok, hi, mid),
                        jnp.where(ok, mid, t), jnp.where(ok, cm, c), it + 1)

            _, _, t, c, _ = lax.while_loop(
                w_cond, w_body, (t0, rmax, t0, c0, jnp.int32(0)))

            off = jnp.int32(0)
            for j in range(NBLK // 16):
                v = bmb[i, pl.ds(j * 16, 16)]
                m = v >= t
                plsc.store_compressed(candb.at[pl.ds(i * CSLOT + off, 16)],
                                      iota + j * 16, mask=m)
                off = jnp.minimum(off + popcnt(m), CMAX)
            ccount_s[i] = off
            thr_s[i] = t
            gbase = (row0 + i) * NPAIR
            for s in range(CMAX // 16):
                f = i * CMAX + s * 16
                gidx[f // 128, pl.ds(f % 128, 16)] = (
                    (candb[pl.ds(i * CSLOT + s * 16, 16)] >> 2) + gbase)
            return c1

        lax.fori_loop(0, BATCH, p1_row, jnp.int32(0))

        descs = [pltpu.async_copy(scores_hbm.at[gidx.at[jj]],
                                  gath.at[pl.ds(jj * 128, 128)], sem_g)
                 for jj in range(BATCH * CMAX // 128)]
        for dsc in descs:
            dsc.wait()

        def p2_row(i, c2):
            cc = ccount_s[i]
            t = thr_s[i]

            def grp(g, off_g):
                cvreg = candb[pl.ds(i * CSLOT + g * 16, 16)]
                validm = (g * 16 + iota) < cc
                slotv = i * CMAX + g * 16 + iota
                colbase = (cvreg & 3) * SBLK
                gidbase = cvreg * SBLK
                o = off_g
                for s2 in range(SBLK):
                    vals = plsc.load_gather(gath, [slotv, colbase + s2])
                    m = jnp.logical_and(vals >= t, validm)
                    gidv = gidbase + s2
                    plsc.store_compressed(selv.at[pl.ds(o, 16)], vals,
                                          mask=m)
                    plsc.store_compressed(selg.at[pl.ds(o, 16)], gidv,
                                          mask=m)
                    o = jnp.minimum(o + popcnt(m), SELCAP)
                return o

            cnt = lax.fori_loop(0, (cc + 15) // 16, grp, jnp.int32(0))

            negv = jnp.full((16,), NEG, jnp.float32)
            zv2 = jnp.zeros((16,), jnp.int32)

            def mbody(iv, st):
                tlo, glo, thi, ghi = st
                x = selv[pl.ds(iv * 16, 16)]
                gx = selg[pl.ds(iv * 16, 16)]
                x = jnp.where(iv * 16 + iota < cnt, x, NEG)
                sx, sgx = plsc.sort_key_val(x, gx)
                rs = lax.rev(sx, (0,))
                rgs = lax.rev(sgx, (0,))
                m1 = tlo >= rs
                u = jnp.where(m1, tlo, rs)
                ug = jnp.where(m1, glo, rgs)
                us, ugs = plsc.sort_key_val(u, ug)
                ru = lax.rev(us, (0,))
                rug = lax.rev(ugs, (0,))
                m2 = thi >= ru
                hi = jnp.where(m2, thi, ru)
                ghi2 = jnp.where(m2, ghi, rug)
                lo = jnp.where(m2, ru, thi)
                glo2 = jnp.where(m2, rug, ghi)
                his, ghis = plsc.sort_key_val(hi, ghi2)
                los, glos = plsc.sort_key_val(lo, glo2)
                return (los, glos, his, ghis)

            nv = (cnt + 15) // 16
            tlo, glo, thi, ghi = lax.fori_loop(
                0, nv, mbody, (negv, zv2, negv, zv2))

            hsst[i, pl.ds(0, 16)] = tlo
            hsst[i, pl.ds(16, 16)] = thi
            f = i * K
            mvidx[f // 128, pl.ds(f % 128, 16)] = glo
            mvidx[f // 128, pl.ds(f % 128 + 16, 16)] = ghi
            return c2

        lax.fori_loop(0, BATCH, p2_row, jnp.int32(0))

        descs = [pltpu.async_copy(memv_hbm.at[mvidx.at[jj]],
                                  gvst.at[pl.ds(jj * 128, 128)], sem_v)
                 for jj in range(BATCH * K // 128)]
        for dsc in descs:
            dsc.wait()

        def hs_fix(i, c3):
            base = i * K
            kn_lo = plsc.load_gather(gvst, [base + iota, kncol])
            kn_hi = plsc.load_gather(gvst, [base + 16 + iota, kncol])
            hsst[i, pl.ds(0, 16)] = hsst[i, pl.ds(0, 16)] * kn_lo
            hsst[i, pl.ds(16, 16)] = hsst[i, pl.ds(16, 16)] * kn_hi
            return c3

        lax.fori_loop(0, BATCH, hs_fix, jnp.int32(0))
        pltpu.sync_copy(hsst, hs_hbm.at[pl.ds(row0, BATCH)])
        pltpu.sync_copy(gvst, gv_hbm.at[pl.ds(row0 * K, BATCH * K)])
        return carry

    lax.fori_loop(0, nbatch, batch_body, jnp.int32(0))


def _sc_topk(scores, bm, memv_pad, hg):
    rows = hg * N
    rpw = rows // NW
    nbatch = rpw // BATCH
    scores_pairs = scores.reshape(rows * NPAIR, GBLK)
    bm2 = bm
    mesh = plsc.VectorSubcoreMesh(core_axis_name="c", subcore_axis_name="s",
                                  num_cores=2, num_subcores=16)
    kern = pl.kernel(
        functools.partial(_sc_topk_body, rpw, nbatch),
        out_type=[jax.ShapeDtypeStruct((rows, K), jnp.float32),
                  jax.ShapeDtypeStruct((rows * K, 2 * DH), jnp.float32)],
        mesh=mesh,
        compiler_params=pltpu.CompilerParams(needs_layout_passes=False),
        scratch_types=[
            pltpu.VMEM((BATCH, NBLK), jnp.float32),
            pltpu.VMEM((BATCH * CMAX // 128, 128), jnp.int32),
            pltpu.VMEM((BATCH * CMAX, GBLK), jnp.float32),
            pltpu.VMEM((BATCH * CSLOT,), jnp.int32),
            pltpu.VMEM((SELCAP + 32,), jnp.float32),
            pltpu.VMEM((SELCAP + 32,), jnp.int32),
            pltpu.VMEM((BATCH, K), jnp.float32),
            pltpu.VMEM((BATCH * K // 128, 128), jnp.int32),
            pltpu.VMEM((BATCH * K, 2 * DH), jnp.float32),
            pltpu.SMEM((BATCH,), jnp.int32),
            pltpu.SMEM((BATCH,), jnp.float32),
            pltpu.SemaphoreType.DMA,
            pltpu.SemaphoreType.DMA,
        ])
    hs, gv = kern(scores_pairs, bm2, memv_pad)
    return hs.reshape(hg, N, K), gv.reshape(hg, N, K, 2 * DH)


def kernel(x, mem_k, mem_v, params):
    x2 = x[0]
    mem_k2 = mem_k[0]
    mem_v2 = mem_v[0]
    layers = params["layers"]

    def heads(t):
        return t.reshape(N, H, DH).transpose(1, 0, 2)

    def unheads(t):
        return t.transpose(1, 0, 2).reshape(N, D)

    p = layers[0]
    q, k, v = _qkv(x2, p["ln1_g"], p["ln1_b"], p["Wq"], p["Wk"], p["Wv"])
    a0 = _attn0(heads(q), heads(k), heads(v))
    h = _proj_residual(x2, unheads(a0), p["Wo"])
    h = _ffn(h, p["ln2_g"], p["ln2_b"], p["W1"], p["b1"], p["W2"], p["b2"])

    p = layers[1]
    q, k, v = _qkv(h, p["ln1_g"], p["ln1_b"], p["Wq"], p["Wk"], p["Wv"])
    qh, kh, vh = heads(q), heads(k), heads(v)
    qn, qfac = _qnorm(qh)
    mkn, memv_pad = _memnorm(mem_k2, mem_v2)
    gate = jax.nn.sigmoid(p["gate"]).reshape(H, 1, 1)
    ngrp = 2
    hg = H // ngrp
    a1_parts = []
    for g in range(ngrp):
        sl = slice(g * hg, (g + 1) * hg)
        scores_g, bm_g = _scores(qn[sl], mkn, hg)
        hs_g, gv_g = _sc_topk(scores_g, bm_g, memv_pad, hg)
        a1_parts.append(_memattn(qh[sl], kh[sl], vh[sl], hs_g,
                                 qfac[sl], gv_g, gate[sl]))
    a1 = jnp.concatenate(a1_parts, axis=0)
    h = _proj_residual(h, unheads(a1), p["Wo"])
    h = _ffn(h, p["ln2_g"], p["ln2_b"], p["W1"], p["b1"], p["W2"], p["b2"])

    out = _final_ln(h, params["lnf_g"], params["lnf_b"])
    return out[None]


# speedup vs baseline: 1.3731x; 1.3731x over previous
import functools

import jax
import jax.numpy as jnp
from jax import lax
from jax.experimental import pallas as pl
from jax.experimental.pallas import tpu as pltpu
from jax.experimental.pallas import tpu_sc as plsc

N, D = 2048, 768
H, DH = 12, 64
M, K = 32768, 32
FF = 4 * D
SCALE = DH ** -0.5
NT = 256
QT = 512


def _lnf(x, g, b):
    m = x.mean(-1, keepdims=True)
    v = ((x - m) ** 2).mean(-1, keepdims=True)
    return (x - m) / jnp.sqrt(v + 1e-5) * g + b


def _gelu(x):
    c = 0.7978845608028654
    return 0.5 * x * (1.0 + jnp.tanh(c * (x + 0.044715 * x * x * x)))


def _dot(a, b):
    return jax.lax.dot_general(a, b, (((1,), (0,)), ((), ())),
                               preferred_element_type=jnp.float32)


def _dot_t(a, b):
    return jax.lax.dot_general(a, b, (((1,), (1,)), ((), ())),
                               preferred_element_type=jnp.float32)


def _qkv_body(x_ref, g_ref, b_ref, wq_ref, wk_ref, wv_ref,
              q_ref, k_ref, v_ref):
    y = _lnf(x_ref[...], g_ref[...], b_ref[...])
    q_ref[...] = _dot(y, wq_ref[...])
    k_ref[...] = _dot(y, wk_ref[...])
    v_ref[...] = _dot(y, wv_ref[...])


def _qkv(x, g, b, wq, wk, wv):
    grid = (N // NT,)
    bs_x = pl.BlockSpec((NT, D), lambda i: (i, 0))
    bs_vec = pl.BlockSpec((1, D), lambda i: (0, 0))
    bs_w = pl.BlockSpec((D, D), lambda i: (0, 0))
    bs_o = pl.BlockSpec((NT, D), lambda i: (i, 0))
    out = pl.pallas_call(
        _qkv_body,
        grid=grid,
        in_specs=[bs_x, bs_vec, bs_vec, bs_w, bs_w, bs_w],
        out_specs=[bs_o, bs_o, bs_o],
        out_shape=[jax.ShapeDtypeStruct((N, D), jnp.float32)] * 3,
    )(x, g.reshape(1, D), b.reshape(1, D), wq, wk, wv)
    return out


def _attn0_body(q_ref, k_ref, v_ref, o_ref):
    qi = pl.program_id(1)
    q = q_ref[0]
    s = _dot_t(q, k_ref[0]) * SCALE
    row = qi * QT + jax.lax.broadcasted_iota(jnp.int32, (QT, N), 0)
    col = jax.lax.broadcasted_iota(jnp.int32, (QT, N), 1)
    s = jnp.where(row >= col, s, -1e9)
    m = jnp.max(s, axis=-1, keepdims=True)
    e = jnp.exp(s - m)
    a = e / jnp.sum(e, axis=-1, keepdims=True)
    o_ref[0] = _dot(a, v_ref[0])


def _attn0(qh, kh, vh):
    grid = (H, N // QT)
    bs_q = pl.BlockSpec((1, QT, DH), lambda h, i: (h, i, 0))
    bs_kv = pl.BlockSpec((1, N, DH), lambda h, i: (h, 0, 0))
    return pl.pallas_call(
        _attn0_body,
        grid=grid,
        in_specs=[bs_q, bs_kv, bs_kv],
        out_specs=bs_q,
        out_shape=jax.ShapeDtypeStruct((H, N, DH), jnp.float32),
    )(qh, kh, vh)


def _proj_body(h_ref, a_ref, w_ref, o_ref):
    o_ref[...] = h_ref[...] + _dot(a_ref[...], w_ref[...])


def _proj_residual(h, a, wo):
    grid = (N // NT,)
    bs = pl.BlockSpec((NT, D), lambda i: (i, 0))
    bs_w = pl.BlockSpec((D, D), lambda i: (0, 0))
    return pl.pallas_call(
        _proj_body,
        grid=grid,
        in_specs=[bs, bs, bs_w],
        out_specs=bs,
        out_shape=jax.ShapeDtypeStruct((N, D), jnp.float32),
    )(h, a, wo)


def _ffn_body(h_ref, g_ref, b_ref, w1_ref, b1_ref, w2_ref, b2_ref, o_ref):
    h = h_ref[...]
    y = _lnf(h, g_ref[...], b_ref[...])
    u = _gelu(_dot(y, w1_ref[...]) + b1_ref[...])
    o_ref[...] = h + _dot(u, w2_ref[...]) + b2_ref[...]


def _ffn(h, g, b, w1, b1, w2, b2):
    grid = (N // NT,)
    bs = pl.BlockSpec((NT, D), lambda i: (i, 0))
    bs_vd = pl.BlockSpec((1, D), lambda i: (0, 0))
    bs_vf = pl.BlockSpec((1, FF), lambda i: (0, 0))
    bs_w1 = pl.BlockSpec((D, FF), lambda i: (0, 0))
    bs_w2 = pl.BlockSpec((FF, D), lambda i: (0, 0))
    return pl.pallas_call(
        _ffn_body,
        grid=grid,
        in_specs=[bs, bs_vd, bs_vd, bs_w1, bs_vf, bs_w2, bs_vd],
        out_specs=bs,
        out_shape=jax.ShapeDtypeStruct((N, D), jnp.float32),
    )(h, g.reshape(1, D), b.reshape(1, D), w1, b1.reshape(1, FF),
      w2, b2.reshape(1, D))


def _ln_body(h_ref, g_ref, b_ref, o_ref):
    o_ref[...] = _lnf(h_ref[...], g_ref[...], b_ref[...])


def _final_ln(h, g, b):
    grid = (N // NT,)
    bs = pl.BlockSpec((NT, D), lambda i: (i, 0))
    bs_vec = pl.BlockSpec((1, D), lambda i: (0, 0))
    return pl.pallas_call(
        _ln_body,
        grid=grid,
        in_specs=[bs, bs_vec, bs_vec],
        out_specs=bs,
        out_shape=jax.ShapeDtypeStruct((N, D), jnp.float32),
    )(h, g.reshape(1, D), b.reshape(1, D))


def _memnorm_body(mk_ref, mv_ref, mkn_ref, mvp_ref):
    mk = mk_ref[...]
    nrm = jnp.sqrt(jnp.sum(mk * mk, axis=-1, keepdims=True)) + 1e-8
    mkn_ref[...] = mk / nrm
    mv = mv_ref[...]
    mvp_ref[...] = jnp.concatenate(
        [mv, nrm, jnp.zeros_like(mv[:, :DH - 1])], axis=-1)


def _memnorm(mem_k, mem_v):
    MT = 4096
    grid = (M // MT,)
    bs = pl.BlockSpec((MT, DH), lambda i: (i, 0))
    bs_p = pl.BlockSpec((MT, 2 * DH), lambda i: (i, 0))
    return pl.pallas_call(
        _memnorm_body,
        grid=grid,
        in_specs=[bs, bs],
        out_specs=[bs, bs_p],
        out_shape=[jax.ShapeDtypeStruct((M, DH), jnp.float32),
                   jax.ShapeDtypeStruct((M, 2 * DH), jnp.float32)],
    )(mem_k, mem_v)


def _qnorm_body(q_ref, qn_ref, qf_ref):
    q = q_ref[0]
    nrm = jnp.sqrt(jnp.sum(q * q, axis=-1, keepdims=True)) + 1e-8
    qn_ref[0] = q / nrm
    qf_ref[0] = nrm


def _qnorm(qh):
    grid = (H,)
    bs = pl.BlockSpec((1, N, DH), lambda h: (h, 0, 0))
    bs_n = pl.BlockSpec((1, N, 1), lambda h: (h, 0, 0))
    return pl.pallas_call(
        _qnorm_body,
        grid=grid,
        in_specs=[bs],
        out_specs=[bs, bs_n],
        out_shape=[jax.ShapeDtypeStruct((H, N, DH), jnp.float32),
                   jax.ShapeDtypeStruct((H, N, 1), jnp.float32)],
    )(qh)


MT_S = 2048
GBLK = 128
SBLK = 32
NBLK = M // SBLK
RR = H * N


def _scores_body(qn_ref, mkn_ref, s_ref, bm_ref):
    q = qn_ref[0]
    parts = []
    for j in range(MT_S // GBLK):
        s_j = _dot_t(q, mkn_ref[pl.ds(j * GBLK, GBLK), :])
        s_ref[j] = s_j
        for u in range(GBLK // SBLK):
            parts.append(jnp.max(s_j[:, u * SBLK:(u + 1) * SBLK], axis=-1,
                                 keepdims=True))
    bm_ref[0] = jnp.concatenate(parts, axis=-1)


def _scores(qn, mkn, hg):
    rows = hg * N
    grid = (hg, M // MT_S)
    bs_q = pl.BlockSpec((1, N, DH), lambda h, m: (h, 0, 0))
    bs_m = pl.BlockSpec((MT_S, DH), lambda h, m: (m, 0))
    bs_s = pl.BlockSpec((MT_S // GBLK, N, GBLK), lambda h, m: (m, h, 0))
    bs_b = pl.BlockSpec((1, N, MT_S // SBLK), lambda h, m: (m, h, 0))
    s3, bm3 = pl.pallas_call(
        _scores_body,
        grid=grid,
        in_specs=[bs_q, bs_m],
        out_specs=[bs_s, bs_b],
        out_shape=[jax.ShapeDtypeStruct((M // GBLK, rows, GBLK), jnp.float32),
                   jax.ShapeDtypeStruct((M // MT_S, rows, MT_S // SBLK),
                                        jnp.float32)],
    )(qn, mkn)
    bm = bm3.transpose(1, 0, 2).reshape(rows, NBLK)
    return s3, bm


def _memattn_body(q_ref, k_ref, v_ref, hs_ref, qf_ref, gv_ref, gate_ref,
                  o_ref):
    qi = pl.program_id(1)
    q = q_ref[0]
    s = _dot_t(q, k_ref[0]) * SCALE
    row = qi * QT + jax.lax.broadcasted_iota(jnp.int32, (QT, N), 0)
    col = jax.lax.broadcasted_iota(jnp.int32, (QT, N), 1)
    s = jnp.where(row >= col, s, -1e9)
    sim_r = hs_ref[0] * qf_ref[0] * SCALE
    m = jnp.maximum(jnp.max(s, axis=-1, keepdims=True),
                    jnp.max(sim_r, axis=-1, keepdims=True))
    e_loc = jnp.exp(s - m)
    e_mem = jnp.exp(sim_r - m)
    denom = jnp.sum(e_loc, axis=-1, keepdims=True) + \
        jnp.sum(e_mem, axis=-1, keepdims=True)
    a_loc = e_loc / denom
    a_mem = e_mem / denom
    out_loc = _dot(a_loc, v_ref[0])
    gv = gv_ref[0]
    acc = a_mem[:, 0:1] * gv[:, 0, :DH]
    for kk in range(1, K):
        acc = acc + a_mem[:, kk:kk + 1] * gv[:, kk, :DH]
    g = gate_ref[0, 0, 0]
    o_ref[0] = g * acc + (1.0 - g) * out_loc


def _memattn(qh, kh, vh, halfsim, qfac, gv, gate):
    hg = qh.shape[0]
    grid = (hg, N // QT)
    bs_q = pl.BlockSpec((1, QT, DH), lambda h, i: (h, i, 0))
    bs_kv = pl.BlockSpec((1, N, DH), lambda h, i: (h, 0, 0))
    bs_hs = pl.BlockSpec((1, QT, K), lambda h, i: (h, i, 0))
    bs_qf = pl.BlockSpec((1, QT, 1), lambda h, i: (h, i, 0))
    bs_gv = pl.BlockSpec((1, QT, K, 2 * DH), lambda h, i: (h, i, 0, 0))
    bs_g = pl.BlockSpec((1, 1, 1), lambda h, i: (h, 0, 0))
    return pl.pallas_call(
        _memattn_body,
        grid=grid,
        in_specs=[bs_q, bs_kv, bs_kv, bs_hs, bs_qf, bs_gv, bs_g],
        out_specs=bs_q,
        out_shape=jax.ShapeDtypeStruct((hg, N, DH), jnp.float32),
    )(qh, kh, vh, halfsim, qfac, gv, gate)


R = H * N
NW = 32
RPW = R // NW
BATCH = 8
NBATCH = RPW // BATCH
NPAIR = M // GBLK
CMAX = 64
CSLOT = 80
SELCAP = 240
NEG = -3.0e38


def _sc_topk_body(rpw, nbatch, scores_hbm, bm_hbm, memv_hbm, hs_hbm, gv_hbm,
                  bmb, gidx, gath, candb, selv, selg, hsst, mvidx,
                  gvst, ccount_s, thr_s, sem_g, sem_v):
    nc = 2
    wid = lax.axis_index("s") * nc + lax.axis_index("c")
    base_row = wid * rpw
    iota = lax.iota(jnp.int32, 16)
    kncol = jnp.full((16,), DH, jnp.int32)

    def popcnt(m):
        return plsc.all_reduce_population_count(m)[0]

    def count_ge(i, t):
        c = jnp.int32(0)
        for j in range(NBLK // 16):
            c = c + popcnt(bmb[i, pl.ds(j * 16, 16)] >= t)
        return c

    def batch_body(b, carry):
        row0 = base_row + b * BATCH
        pltpu.sync_copy(bm_hbm.at[pl.ds(row0, BATCH)], bmb)
        zv = jnp.zeros((16,), jnp.int32)
        for j in range(BATCH * CSLOT // 16):
            candb[pl.ds(j * 16, 16)] = zv

        def p1_row(i, c1):
            mh = []
            for half in range(2):
                m = bmb[i, pl.ds(half * (NBLK // 2), 16)]
                for j in range(1, NBLK // 32):
                    m = jnp.maximum(
                        m, bmb[i, pl.ds(half * (NBLK // 2) + j * 16, 16)])
                mh.append(m)
            t0 = jnp.sort(jnp.minimum(mh[0], mh[1]))[0]
            rmax = jnp.sort(jnp.maximum(mh[0], mh[1]))[15]
            c0 = count_ge(i, t0)

            def w_cond(st):
                return jnp.logical_and(st[3] > 48, st[4] < 16)

            def w_body(st):
                lo, hi, t, c, it = st
                mid = 0.5 * (lo + hi)
                cm = count_ge(i, mid)
                ok = cm >= K
                return (jnp.where(ok, mid, lo), jnp.where(ok, hi, mid),
                        jnp.where(ok, mid, t), jnp.where(ok, cm, c), it + 1)

            _, _, t, c, _ = lax.while_loop(
                w_cond, w_body, (t0, rmax, t0, c0, jnp.int32(0)))

            off = jnp.int32(0)
            for j in range(NBLK // 16):
                v = bmb[i, pl.ds(j * 16, 16)]
                m = v >= t
                plsc.store_compressed(candb.at[pl.ds(i * CSLOT + off, 16)],
                                      iota + j * 16, mask=m)
                off = jnp.minimum(off + popcnt(m), CMAX)
            ccount_s[i] = off
            thr_s[i] = t
            rows_all = rpw * NW
            gbase = row0 + i
            for s in range(CMAX // 16):
                f = i * CMAX + s * 16
                gidx[f // 128, pl.ds(f % 128, 16)] = (
                    (candb[pl.ds(i * CSLOT + s * 16, 16)] >> 2) * rows_all
                    + gbase)
            return c1

        lax.fori_loop(0, BATCH, p1_row, jnp.int32(0))

        descs = [pltpu.async_copy(scores_hbm.at[gidx.at[jj]],
                                  gath.at[pl.ds(jj * 128, 128)], sem_g)
                 for jj in range(BATCH * CMAX // 128)]
        for dsc in descs:
            dsc.wait()

        def p2_row(i, c2):
            cc = ccount_s[i]
            t = thr_s[i]

            def grp(g, off_g):
                cvreg = candb[pl.ds(i * CSLOT + g * 16, 16)]
                validm = (g * 16 + iota) < cc
                slotv = i * CMAX + g * 16 + iota
                colbase = (cvreg & 3) * SBLK
                gidbase = cvreg * SBLK
                o = off_g
                for s2 in range(SBLK):
                    vals = plsc.load_gather(gath, [slotv, colbase + s2])
                    m = jnp.logical_and(vals >= t, validm)
                    gidv = gidbase + s2
                    plsc.store_compressed(selv.at[pl.ds(o, 16)], vals,
                                          mask=m)
                    plsc.store_compressed(selg.at[pl.ds(o, 16)], gidv,
                                          mask=m)
                    o = jnp.minimum(o + popcnt(m), SELCAP)
                return o

            cnt = lax.fori_loop(0, (cc + 15) // 16, grp, jnp.int32(0))

            negv = jnp.full((16,), NEG, jnp.float32)
            zv2 = jnp.zeros((16,), jnp.int32)

            def mbody(iv, st):
                tlo, glo, thi, ghi = st
                x = selv[pl.ds(iv * 16, 16)]
                gx = selg[pl.ds(iv * 16, 16)]
                x = jnp.where(iv * 16 + iota < cnt, x, NEG)
                sx, sgx = plsc.sort_key_val(x, gx)
                rs = lax.rev(sx, (0,))
                rgs = lax.rev(sgx, (0,))
                m1 = tlo >= rs
                u = jnp.where(m1, tlo, rs)
                ug = jnp.where(m1, glo, rgs)
                us, ugs = plsc.sort_key_val(u, ug)
                ru = lax.rev(us, (0,))
                rug = lax.rev(ugs, (0,))
                m2 = thi >= ru
                hi = jnp.where(m2, thi, ru)
                ghi2 = jnp.where(m2, ghi, rug)
                lo = jnp.where(m2, ru, thi)
                glo2 = jnp.where(m2, rug, ghi)
                his, ghis = plsc.sort_key_val(hi, ghi2)
                los, glos = plsc.sort_key_val(lo, glo2)
                return (los, glos, his, ghis)

            nv = (cnt + 15) // 16
            tlo, glo, thi, ghi = lax.fori_loop(
                0, nv, mbody, (negv, zv2, negv, zv2))

            hsst[i, pl.ds(0, 16)] = tlo
            hsst[i, pl.ds(16, 16)] = thi
            f = i * K
            mvidx[f // 128, pl.ds(f % 128, 16)] = glo
            mvidx[f // 128, pl.ds(f % 128 + 16, 16)] = ghi
            return c2

        lax.fori_loop(0, BATCH, p2_row, jnp.int32(0))

        descs = [pltpu.async_copy(memv_hbm.at[mvidx.at[jj]],
                                  gvst.at[pl.ds(jj * 128, 128)], sem_v)
                 for jj in range(BATCH * K // 128)]
        for dsc in descs:
            dsc.wait()

        def hs_fix(i, c3):
            base = i * K
            kn_lo = plsc.load_gather(gvst, [base + iota, kncol])
            kn_hi = plsc.load_gather(gvst, [base + 16 + iota, kncol])
            hsst[i, pl.ds(0, 16)] = hsst[i, pl.ds(0, 16)] * kn_lo
            hsst[i, pl.ds(16, 16)] = hsst[i, pl.ds(16, 16)] * kn_hi
            return c3

        lax.fori_loop(0, BATCH, hs_fix, jnp.int32(0))
        pltpu.sync_copy(hsst, hs_hbm.at[pl.ds(row0, BATCH)])
        pltpu.sync_copy(gvst, gv_hbm.at[pl.ds(row0 * K, BATCH * K)])
        return carry

    lax.fori_loop(0, nbatch, batch_body, jnp.int32(0))


def _sc_topk(scores, bm, memv_pad, hg):
    rows = hg * N
    rpw = rows // NW
    nbatch = rpw // BATCH
    scores_pairs = scores.reshape(NPAIR * rows, GBLK)
    bm2 = bm
    mesh = plsc.VectorSubcoreMesh(core_axis_name="c", subcore_axis_name="s",
                                  num_cores=2, num_subcores=16)
    kern = pl.kernel(
        functools.partial(_sc_topk_body, rpw, nbatch),
        out_type=[jax.ShapeDtypeStruct((rows, K), jnp.float32),
                  jax.ShapeDtypeStruct((rows * K, 2 * DH), jnp.float32)],
        mesh=mesh,
        compiler_params=pltpu.CompilerParams(needs_layout_passes=False),
        scratch_types=[
            pltpu.VMEM((BATCH, NBLK), jnp.float32),
            pltpu.VMEM((BATCH * CMAX // 128, 128), jnp.int32),
            pltpu.VMEM((BATCH * CMAX, GBLK), jnp.float32),
            pltpu.VMEM((BATCH * CSLOT,), jnp.int32),
            pltpu.VMEM((SELCAP + 32,), jnp.float32),
            pltpu.VMEM((SELCAP + 32,), jnp.int32),
            pltpu.VMEM((BATCH, K), jnp.float32),
            pltpu.VMEM((BATCH * K // 128, 128), jnp.int32),
            pltpu.VMEM((BATCH * K, 2 * DH), jnp.float32),
            pltpu.SMEM((BATCH,), jnp.int32),
            pltpu.SMEM((BATCH,), jnp.float32),
            pltpu.SemaphoreType.DMA,
            pltpu.SemaphoreType.DMA,
        ])
    hs, gv = kern(scores_pairs, bm2, memv_pad)
    return hs.reshape(hg, N, K), gv.reshape(hg, N, K, 2 * DH)


def kernel(x, mem_k, mem_v, params):
    x2 = x[0]
    mem_k2 = mem_k[0]
    mem_v2 = mem_v[0]
    layers = params["layers"]

    def heads(t):
        return t.reshape(N, H, DH).transpose(1, 0, 2)

    def unheads(t):
        return t.transpose(1, 0, 2).reshape(N, D)

    p = layers[0]
    q, k, v = _qkv(x2, p["ln1_g"], p["ln1_b"], p["Wq"], p["Wk"], p["Wv"])
    a0 = _attn0(heads(q), heads(k), heads(v))
    h = _proj_residual(x2, unheads(a0), p["Wo"])
    h = _ffn(h, p["ln2_g"], p["ln2_b"], p["W1"], p["b1"], p["W2"], p["b2"])

    p = layers[1]
    q, k, v = _qkv(h, p["ln1_g"], p["ln1_b"], p["Wq"], p["Wk"], p["Wv"])
    qh, kh, vh = heads(q), heads(k), heads(v)
    qn, qfac = _qnorm(qh)
    mkn, memv_pad = _memnorm(mem_k2, mem_v2)
    gate = jax.nn.sigmoid(p["gate"]).reshape(H, 1, 1)
    ngrp = 2
    hg = H // ngrp
    a1_parts = []
    for g in range(ngrp):
        sl = slice(g * hg, (g + 1) * hg)
        scores_g, bm_g = _scores(qn[sl], mkn, hg)
        hs_g, gv_g = _sc_topk(scores_g, bm_g, memv_pad, hg)
        a1_parts.append(_memattn(qh[sl], kh[sl], vh[sl], hs_g,
                                 qfac[sl], gv_g, gate[sl]))
    a1 = jnp.concatenate(a1_parts, axis=0)
    h = _proj_residual(h, unheads(a1), p["Wo"])
    h = _ffn(h, p["ln2_g"], p["ln2_b"], p["W1"], p["b1"], p["W2"], p["b2"])

    out = _final_ln(h, params["lnf_g"], params["lnf_b"])
    return out[None]
